# baseline (speedup 1.0000x reference)
import jax
import jax.numpy as jnp
from jax.experimental import pallas as pl
from jax.experimental.pallas import tpu as pltpu

_DIN = 1024
_DOUT = 1024
_E = 8
_GK = 8
_K = 32
_GATE = _E * _GK
_RANK = _E * _K
_TN = 1024


def _split3(a):
    hi = a.astype(jnp.bfloat16)
    r1 = a - hi.astype(jnp.float32)
    mid = r1.astype(jnp.bfloat16)
    lo = (r1 - mid.astype(jnp.float32)).astype(jnp.bfloat16)
    return hi, mid, lo


def _fused_kernel(x_ref, wpb_ref, gtb_ref, mtb_ref, uf_ref, embeb_ref,
                  o_ref):
    x = x_ref[...]
    xb = x.astype(jnp.bfloat16)
    _dnt = (((1,), (1,)), ((), ()))
    g = jax.lax.dot_general(xb, gtb_ref[...], _dnt,
                            preferred_element_type=jnp.float32)
    t = jax.lax.dot_general(xb, mtb_ref[...], _dnt,
                            preferred_element_type=jnp.float32)

    gi = jax.lax.broadcasted_iota(jnp.int32, (_GATE, _E), 0) // _GK
    ge = jax.lax.broadcasted_iota(jnp.int32, (_GATE, _E), 1)
    gmat = (gi == ge).astype(jnp.bfloat16)
    gsq = g * g
    sh, sm, sl = _split3(gsq)
    d = lambda u: jnp.dot(u, gmat, preferred_element_type=jnp.float32)
    ss = d(sh) + d(sm) + d(sl)
    logits = jnp.sqrt(ss)

    lt = logits.T
    iota = jax.lax.broadcasted_iota(jnp.int32, lt.shape, 0)
    m1 = jnp.max(lt, axis=0, keepdims=True)
    i1 = jnp.min(jnp.where(lt == m1, iota, _E), axis=0, keepdims=True)
    lm = jnp.where(iota == i1, -jnp.inf, lt)
    m2 = jnp.max(lm, axis=0, keepdims=True)
    i2 = jnp.min(jnp.where(lm == m2, iota, _E), axis=0, keepdims=True)
    p2 = jnp.exp(m2 - m1)
    denom = 1.0 + p2
    wt = jnp.where(iota == i1, 1.0, jnp.where(iota == i2, p2, 0.0)) / denom
    w = wt.T

    wrepeb = jnp.dot(w.astype(jnp.bfloat16), embeb_ref[...],
                     preferred_element_type=jnp.float32)
    tw = (t * wrepeb[:, :_RANK]).astype(jnp.bfloat16)

    out = jax.lax.dot_general(xb, wpb_ref[...], _dnt,
                              preferred_element_type=jnp.float32)
    out += jnp.dot(tw, uf_ref[...], preferred_element_type=jnp.float32)
    out += wrepeb[:, _RANK:]
    o_ref[...] = out


def kernel(hidden_states, Wp, bp, gate_w, U, SVH, Eb):
    x = hidden_states.reshape(-1, _DIN)
    n = x.shape[0]
    gt = gate_w.reshape(_GATE, _DIN).astype(jnp.bfloat16)
    mt = SVH.reshape(_RANK, _DIN).astype(jnp.bfloat16)
    wpb = Wp.astype(jnp.bfloat16)
    uf = jnp.transpose(U, (0, 2, 1)).reshape(_RANK, _DOUT).astype(jnp.bfloat16)
    ri = jax.lax.broadcasted_iota(jnp.int32, (_E, _RANK), 0)
    rc = jax.lax.broadcasted_iota(jnp.int32, (_E, _RANK), 1) // _K
    embeb = jnp.concatenate(
        [(ri == rc).astype(jnp.float32), Eb + bp[None, :]], axis=1
    ).astype(jnp.bfloat16)
    out = pl.pallas_call(
        _fused_kernel,
        grid=(n // _TN,),
        in_specs=[
            pl.BlockSpec((_TN, _DIN), lambda i: (i, 0)),
            pl.BlockSpec((_DOUT, _DIN), lambda i: (0, 0)),
            pl.BlockSpec((_GATE, _DIN), lambda i: (0, 0)),
            pl.BlockSpec((_RANK, _DIN), lambda i: (0, 0)),
            pl.BlockSpec((_RANK, _DOUT), lambda i: (0, 0)),
            pl.BlockSpec((_E, _RANK + _DOUT), lambda i: (0, 0)),
        ],
        out_specs=pl.BlockSpec((_TN, _DOUT), lambda i: (i, 0)),
        out_shape=jax.ShapeDtypeStruct((n, _DOUT), jnp.float32),
        compiler_params=pltpu.CompilerParams(
            dimension_semantics=("parallel",),
        ),
    )(x, wpb, gt, mt, uf, embeb)
    return out.reshape(*hidden_states.shape[:-1], _DOUT)


# speedup vs baseline: 1.2022x; 1.2022x over previous
import jax
import jax.numpy as jnp
from jax.experimental import pallas as pl
from jax.experimental.pallas import tpu as pltpu

_DIN = 1024
_DOUT = 1024
_E = 8
_GK = 8
_K = 32
_GATE = _E * _GK
_RANK = _E * _K
_TN = 1024


def _split3(a):
    hi = a.astype(jnp.bfloat16)
    r1 = a - hi.astype(jnp.float32)
    mid = r1.astype(jnp.bfloat16)
    lo = (r1 - mid.astype(jnp.float32)).astype(jnp.bfloat16)
    return hi, mid, lo


def _fused_kernel(x_ref, wp_ref, gt_ref, mt_ref, uf_ref, embeb_ref,
                  o_ref, wpb_ref, gtb_ref, mtb_ref):
    @pl.when(pl.program_id(0) == 0)
    def _cast_weights():
        wpb_ref[...] = wp_ref[...].astype(jnp.bfloat16)
        gtb_ref[...] = gt_ref[...].astype(jnp.bfloat16)
        mtb_ref[...] = mt_ref[...].astype(jnp.bfloat16)

    x = x_ref[...]
    xb = x.astype(jnp.bfloat16)
    _dnt = (((1,), (1,)), ((), ()))
    g = jax.lax.dot_general(xb, gtb_ref[...], _dnt,
                            preferred_element_type=jnp.float32)
    t = jax.lax.dot_general(xb, mtb_ref[...], _dnt,
                            preferred_element_type=jnp.float32)

    gi = jax.lax.broadcasted_iota(jnp.int32, (_GATE, _E), 0) // _GK
    ge = jax.lax.broadcasted_iota(jnp.int32, (_GATE, _E), 1)
    gmat = (gi == ge).astype(jnp.bfloat16)
    gsq = g * g
    sh, sm, sl = _split3(gsq)
    d = lambda u: jnp.dot(u, gmat, preferred_element_type=jnp.float32)
    ss = d(sh) + d(sm) + d(sl)
    logits = jnp.sqrt(ss)

    lt = logits.T
    iota = jax.lax.broadcasted_iota(jnp.int32, lt.shape, 0)
    m1 = jnp.max(lt, axis=0, keepdims=True)
    i1 = jnp.min(jnp.where(lt == m1, iota, _E), axis=0, keepdims=True)
    lm = jnp.where(iota == i1, -jnp.inf, lt)
    m2 = jnp.max(lm, axis=0, keepdims=True)
    i2 = jnp.min(jnp.where(lm == m2, iota, _E), axis=0, keepdims=True)
    p2 = jnp.exp(m2 - m1)
    denom = 1.0 + p2
    wt = jnp.where(iota == i1, 1.0, jnp.where(iota == i2, p2, 0.0)) / denom
    w = wt.T

    wrepeb = jnp.dot(w.astype(jnp.bfloat16), embeb_ref[...],
                     preferred_element_type=jnp.float32)
    tw = (t * wrepeb[:, :_RANK]).astype(jnp.bfloat16)

    out = jax.lax.dot_general(xb, wpb_ref[...], _dnt,
                              preferred_element_type=jnp.float32)
    out += jnp.dot(tw, uf_ref[...], preferred_element_type=jnp.float32)
    out += wrepeb[:, _RANK:]
    o_ref[...] = out


def kernel(hidden_states, Wp, bp, gate_w, U, SVH, Eb):
    x = hidden_states.reshape(-1, _DIN)
    n = x.shape[0]
    gt = gate_w.reshape(_GATE, _DIN)
    mt = SVH.reshape(_RANK, _DIN)
    uf = jnp.transpose(U, (0, 2, 1)).reshape(_RANK, _DOUT).astype(jnp.bfloat16)
    ri = jax.lax.broadcasted_iota(jnp.int32, (_E, _RANK), 0)
    rc = jax.lax.broadcasted_iota(jnp.int32, (_E, _RANK), 1) // _K
    embeb = jnp.concatenate(
        [(ri == rc).astype(jnp.float32), Eb + bp[None, :]], axis=1
    ).astype(jnp.bfloat16)
    out = pl.pallas_call(
        _fused_kernel,
        grid=(n // _TN,),
        in_specs=[
            pl.BlockSpec((_TN, _DIN), lambda i: (i, 0)),
            pl.BlockSpec((_DOUT, _DIN), lambda i: (0, 0)),
            pl.BlockSpec((_GATE, _DIN), lambda i: (0, 0)),
            pl.BlockSpec((_RANK, _DIN), lambda i: (0, 0)),
            pl.BlockSpec((_RANK, _DOUT), lambda i: (0, 0)),
            pl.BlockSpec((_E, _RANK + _DOUT), lambda i: (0, 0)),
        ],
        out_specs=pl.BlockSpec((_TN, _DOUT), lambda i: (i, 0)),
        out_shape=jax.ShapeDtypeStruct((n, _DOUT), jnp.float32),
        scratch_shapes=[
            pltpu.VMEM((_DOUT, _DIN), jnp.bfloat16),
            pltpu.VMEM((_GATE, _DIN), jnp.bfloat16),
            pltpu.VMEM((_RANK, _DIN), jnp.bfloat16),
        ],
    )(x, Wp, gt, mt, uf, embeb)
    return out.reshape(*hidden_states.shape[:-1], _DOUT)


# speedup vs baseline: 1.2609x; 1.0488x over previous
import jax
import jax.numpy as jnp
from jax.experimental import pallas as pl
from jax.experimental.pallas import tpu as pltpu

_DIN = 1024
_DOUT = 1024
_E = 8
_GK = 8
_K = 32
_GATE = _E * _GK
_RANK = _E * _K
_TN = 1024


def _split3(a):
    hi = a.astype(jnp.bfloat16)
    r1 = a - hi.astype(jnp.float32)
    mid = r1.astype(jnp.bfloat16)
    lo = (r1 - mid.astype(jnp.float32)).astype(jnp.bfloat16)
    return hi, mid, lo


def _fused_kernel(x_ref, wp_ref, gt_ref, mt_ref, uf_ref, embeb_ref,
                  o_ref, wpb_ref, gtb_ref, mtb_ref):
    @pl.when(pl.program_id(0) == 0)
    def _cast_weights():
        wpb_ref[...] = wp_ref[...].astype(jnp.bfloat16)
        gtb_ref[...] = gt_ref[...].astype(jnp.bfloat16)
        mtb_ref[...] = mt_ref[...].astype(jnp.bfloat16)

    x = x_ref[...]
    xb = x.astype(jnp.bfloat16)
    _dnt = (((1,), (1,)), ((), ()))
    g = jax.lax.dot_general(xb, gtb_ref[...], _dnt,
                            preferred_element_type=jnp.float32)
    t = jax.lax.dot_general(xb, mtb_ref[...], _dnt,
                            preferred_element_type=jnp.float32)

    gi = jax.lax.broadcasted_iota(jnp.int32, (_GATE, _E), 0) // _GK
    ge = jax.lax.broadcasted_iota(jnp.int32, (_GATE, _E), 1)
    gmat = (gi == ge).astype(jnp.bfloat16)
    gsq = g * g
    sh, sm, sl = _split3(gsq)
    d = lambda u: jnp.dot(u, gmat, preferred_element_type=jnp.float32)
    ss = d(sh) + d(sm) + d(sl)
    logits = jnp.sqrt(ss)

    lt = logits.T
    iota = jax.lax.broadcasted_iota(jnp.int32, lt.shape, 0)
    m1 = jnp.max(lt, axis=0, keepdims=True)
    i1 = jnp.min(jnp.where(lt == m1, iota, _E), axis=0, keepdims=True)
    lm = jnp.where(iota == i1, -jnp.inf, lt)
    m2 = jnp.max(lm, axis=0, keepdims=True)
    i2 = jnp.min(jnp.where(lm == m2, iota, _E), axis=0, keepdims=True)
    p2 = jnp.exp(m2 - m1)
    denom = 1.0 + p2
    wt = jnp.where(iota == i1, 1.0, jnp.where(iota == i2, p2, 0.0)) / denom
    w = wt.T

    wrepeb = jnp.dot(w.astype(jnp.bfloat16), embeb_ref[...],
                     preferred_element_type=jnp.float32)
    tw = (t * wrepeb[:, :_RANK]).astype(jnp.bfloat16)

    out = jax.lax.dot_general(xb, wpb_ref[...], _dnt,
                              preferred_element_type=jnp.float32)
    out += jnp.dot(tw, uf_ref[...], preferred_element_type=jnp.float32)
    out += wrepeb[:, _RANK:]
    o_ref[...] = out


def kernel(hidden_states, Wp, bp, gate_w, U, SVH, Eb):
    x = hidden_states.reshape(-1, _DIN)
    n = x.shape[0]
    gt = gate_w.reshape(_GATE, _DIN)
    mt = SVH.reshape(_RANK, _DIN)
    uf = jnp.transpose(U, (0, 2, 1)).reshape(_RANK, _DOUT).astype(jnp.bfloat16)
    ri = jax.lax.broadcasted_iota(jnp.int32, (_E, _RANK), 0)
    rc = jax.lax.broadcasted_iota(jnp.int32, (_E, _RANK), 1) // _K
    embeb = jnp.concatenate(
        [(ri == rc).astype(jnp.float32), Eb + bp[None, :]], axis=1
    ).astype(jnp.bfloat16)
    out = pl.pallas_call(
        _fused_kernel,
        grid=(n // _TN,),
        in_specs=[
            pl.BlockSpec((_TN, _DIN), lambda i: (i, 0)),
            pl.BlockSpec((_DOUT, _DIN), lambda i: (0, 0)),
            pl.BlockSpec((_GATE, _DIN), lambda i: (0, 0)),
            pl.BlockSpec((_RANK, _DIN), lambda i: (0, 0)),
            pl.BlockSpec((_RANK, _DOUT), lambda i: (0, 0)),
            pl.BlockSpec((_E, _RANK + _DOUT), lambda i: (0, 0)),
        ],
        out_specs=pl.BlockSpec((_TN, _DOUT), lambda i: (i, 0)),
        out_shape=jax.ShapeDtypeStruct((n, _DOUT), jnp.float32),
        compiler_params=pltpu.CompilerParams(
            allow_input_fusion=[False, False, False, False, True, True],
        ),
        scratch_shapes=[
            pltpu.VMEM((_DOUT, _DIN), jnp.bfloat16),
            pltpu.VMEM((_GATE, _DIN), jnp.bfloat16),
            pltpu.VMEM((_RANK, _DIN), jnp.bfloat16),
        ],
    )(x, Wp, gt, mt, uf, embeb)
    return out.reshape(*hidden_states.shape[:-1], _DOUT)


# speedup vs baseline: 1.2703x; 1.0074x over previous
import jax
import jax.numpy as jnp
from jax.experimental import pallas as pl
from jax.experimental.pallas import tpu as pltpu

_DIN = 1024
_DOUT = 1024
_E = 8
_GK = 8
_K = 32
_GATE = _E * _GK
_RANK = _E * _K
_TN = 1024


def _split3(a):
    hi = a.astype(jnp.bfloat16)
    r1 = a - hi.astype(jnp.float32)
    mid = r1.astype(jnp.bfloat16)
    lo = (r1 - mid.astype(jnp.float32)).astype(jnp.bfloat16)
    return hi, mid, lo


def _fused_kernel(x_ref, wp_ref, gt_ref, mt_ref, uf_ref, embeb_ref,
                  o_ref, wpb_ref, gtb_ref, mtb_ref):
    @pl.when(pl.program_id(0) == 0)
    def _cast_weights():
        wpb_ref[...] = wp_ref[...].astype(jnp.bfloat16)
        gtb_ref[...] = gt_ref[...].astype(jnp.bfloat16)
        mtb_ref[...] = mt_ref[...].astype(jnp.bfloat16)

    x = x_ref[...]
    xb = x.astype(jnp.bfloat16)
    _dnt = (((1,), (1,)), ((), ()))
    g = jax.lax.dot_general(xb, gtb_ref[...], _dnt,
                            preferred_element_type=jnp.float32)
    t = jax.lax.dot_general(xb, mtb_ref[...], _dnt,
                            preferred_element_type=jnp.float32)

    gi = jax.lax.broadcasted_iota(jnp.int32, (_GATE, _E), 0) // _GK
    ge = jax.lax.broadcasted_iota(jnp.int32, (_GATE, _E), 1)
    gmat = (gi == ge).astype(jnp.bfloat16)
    gsq = g * g
    sh, sm, sl = _split3(gsq)
    d = lambda u: jnp.dot(u, gmat, preferred_element_type=jnp.float32)
    ss = d(sh) + d(sm) + d(sl)
    logits = jnp.sqrt(ss)

    lt = logits.T
    iota = jax.lax.broadcasted_iota(jnp.int32, lt.shape, 0)
    m1 = jnp.max(lt, axis=0, keepdims=True)
    i1 = jnp.min(jnp.where(lt == m1, iota, _E), axis=0, keepdims=True)
    lm = jnp.where(iota == i1, -jnp.inf, lt)
    m2 = jnp.max(lm, axis=0, keepdims=True)
    i2 = jnp.min(jnp.where(lm == m2, iota, _E), axis=0, keepdims=True)
    p2 = jnp.exp(m2 - m1)
    denom = 1.0 + p2
    wt = jnp.where(iota == i1, 1.0, jnp.where(iota == i2, p2, 0.0)) / denom
    w = wt.T

    wrepeb = jnp.dot(w.astype(jnp.bfloat16), embeb_ref[...],
                     preferred_element_type=jnp.float32)
    tw = (t * wrepeb[:, :_RANK]).astype(jnp.bfloat16)

    out = jax.lax.dot_general(xb, wpb_ref[...], _dnt,
                              preferred_element_type=jnp.float32)
    out += jnp.dot(tw, uf_ref[...], preferred_element_type=jnp.float32)
    out += wrepeb[:, _RANK:]
    o_ref[...] = out


def kernel(hidden_states, Wp, bp, gate_w, U, SVH, Eb):
    x = hidden_states.reshape(-1, _DIN)
    n = x.shape[0]
    gt = gate_w.reshape(_GATE, _DIN)
    mt = SVH.reshape(_RANK, _DIN)
    uf = jnp.transpose(U, (0, 2, 1)).reshape(_RANK, _DOUT).astype(jnp.bfloat16)
    ri = jax.lax.broadcasted_iota(jnp.int32, (_E, _RANK), 0)
    rc = jax.lax.broadcasted_iota(jnp.int32, (_E, _RANK), 1) // _K
    embeb = jnp.concatenate(
        [(ri == rc).astype(jnp.float32), Eb + bp[None, :]], axis=1
    ).astype(jnp.bfloat16)
    out = pl.pallas_call(
        _fused_kernel,
        grid=(n // _TN,),
        in_specs=[
            pl.BlockSpec((_TN, _DIN), lambda i: (i, 0)),
            pl.BlockSpec((_DOUT, _DIN), lambda i: (0, 0)),
            pl.BlockSpec((_GATE, _DIN), lambda i: (0, 0)),
            pl.BlockSpec((_RANK, _DIN), lambda i: (0, 0)),
            pl.BlockSpec((_RANK, _DOUT), lambda i: (0, 0)),
            pl.BlockSpec((_E, _RANK + _DOUT), lambda i: (0, 0)),
        ],
        out_specs=pl.BlockSpec((_TN, _DOUT), lambda i: (i, 0)),
        out_shape=jax.ShapeDtypeStruct((n, _DOUT), jnp.float32),
        compiler_params=pltpu.CompilerParams(
            allow_input_fusion=[True, True, True, True, True, True],
        ),
        scratch_shapes=[
            pltpu.VMEM((_DOUT, _DIN), jnp.bfloat16),
            pltpu.VMEM((_GATE, _DIN), jnp.bfloat16),
            pltpu.VMEM((_RANK, _DIN), jnp.bfloat16),
        ],
    )(x, Wp, gt, mt, uf, embeb)
    return out.reshape(*hidden_states.shape[:-1], _DOUT)
